# baseline (speedup 1.0000x reference)
import functools

import jax
import jax.numpy as jnp
from jax import lax
from jax.experimental import pallas as pl
from jax.experimental.pallas import tpu as pltpu
from jax.experimental.pallas import tpu_sc as plsc

N = 10000
D = 128
HD = D // 2
E = 320000
CLASSES = 40

NC = 2
NS = 16
CH = 128
CPT = 160
E_PAD = NS * CPT * CH
N_PAD = 10240
RPT = N_PAD // NS
ROW_BLKS = RPT // CH


RQ = 2
EPQ = RQ * CH
NQ = CPT // RQ


def _sc_agg_body(with_deg, x2_hbm, src_hbm, dst_hbm, *refs):
    if with_deg:
        (acc_out, deg_out, src_v, dst_v, rows_v, ones_v, b16_v,
         acc_s, deg_s, sem) = refs
    else:
        (acc_out, src_v, dst_v, rows_v, acc_s, sem) = refs

    c = lax.axis_index("c")
    s = lax.axis_index("s")

    zf = jnp.zeros((16,), jnp.float32)
    of = jnp.ones((16,), jnp.float32)

    def initrow(i, carry):
        for t in range(HD // 16):
            rows_v[i, pl.ds(t * 16, 16)] = zf
        if with_deg:
            b16_v[i, pl.ds(0, 16)] = zf
        return carry

    lax.fori_loop(0, CH, initrow, 0)

    if with_deg:
        def initones(i, carry):
            ones_v[i, pl.ds(0, 16)] = of
            return carry

        lax.fori_loop(0, EPQ, initones, 0)

    def zrow(k, carry):
        r = s * RPT + k * CH
        pltpu.sync_copy(rows_v.at[pl.ds(0, CH)], acc_s.at[pl.ds(r, CH)])
        if with_deg:
            pltpu.sync_copy(b16_v, deg_s.at[pl.ds(r, CH)])
        return carry

    lax.fori_loop(0, ROW_BLKS, zrow, 0)

    pltpu.sync_copy(src_hbm.at[pl.ds(s * NQ, NQ)], src_v)
    pltpu.sync_copy(dst_hbm.at[pl.ds(s * NQ, NQ)], dst_v)

    cN = c * N

    def addrow(q, carry):
        for t in range(EPQ // 16):
            sl = pl.ds(t * 16, 16)
            src_v[q, sl] = src_v[q, sl] + cN
        return carry

    lax.fori_loop(0, NQ, addrow, 0)

    plsc.subcore_barrier()

    pltpu.async_copy(x2_hbm.at[src_v.at[0]], rows_v.at[pl.ds(0, EPQ)], sem)

    def batch(q, carry):
        b = lax.rem(q, 2) * EPQ
        ob = lax.rem(q + 1, 2) * EPQ
        pltpu.make_async_copy(
            x2_hbm.at[pl.ds(0, EPQ)], rows_v.at[pl.ds(b, EPQ)], sem).wait()

        @pl.when(q + 1 < NQ)
        def _():
            pltpu.async_copy(
                x2_hbm.at[src_v.at[q + 1]], rows_v.at[pl.ds(ob, EPQ)], sem)

        pltpu.sync_copy(rows_v.at[pl.ds(b, EPQ)], acc_s.at[dst_v.at[q]],
                        add=True)
        if with_deg:
            @pl.when((q >= c * (NQ // 2)) & (q < (c + 1) * (NQ // 2)))
            def _():
                pltpu.sync_copy(ones_v, deg_s.at[dst_v.at[q]], add=True)
        return carry

    lax.fori_loop(0, NQ, batch, 0)

    plsc.subcore_barrier()

    def wrow(k, carry):
        r = s * RPT + k * CH
        pltpu.sync_copy(acc_s.at[pl.ds(r, CH)], rows_v.at[pl.ds(0, CH)])
        pltpu.sync_copy(rows_v.at[pl.ds(0, CH)],
                        acc_out.at[pl.ds(c * N_PAD + r, CH)])
        if with_deg:
            pltpu.sync_copy(deg_s.at[pl.ds(r, CH)], b16_v)
            pltpu.sync_copy(b16_v, deg_out.at[pl.ds(c * N_PAD + r, CH)])
        return carry

    lax.fori_loop(0, ROW_BLKS, wrow, 0)


def _make_sc_agg(with_deg):
    mesh = plsc.VectorSubcoreMesh(
        core_axis_name="c", subcore_axis_name="s", num_cores=NC, num_subcores=NS)
    out_type = [jax.ShapeDtypeStruct((NC * N_PAD, HD), jnp.float32)]
    scratch = [
        pltpu.VMEM((NQ, EPQ), jnp.int32),
        pltpu.VMEM((NQ, EPQ), jnp.int32),
        pltpu.VMEM((2 * EPQ, HD), jnp.float32),
    ]
    if with_deg:
        out_type.append(jax.ShapeDtypeStruct((NC * N_PAD, 16), jnp.float32))
        scratch += [
            pltpu.VMEM((EPQ, 16), jnp.float32),
            pltpu.VMEM((CH, 16), jnp.float32),
        ]
    scratch += [pltpu.VMEM_SHARED((N_PAD, HD), jnp.float32)]
    if with_deg:
        scratch += [pltpu.VMEM_SHARED((N_PAD, 16), jnp.float32)]
    scratch += [pltpu.SemaphoreType.DMA]
    return pl.kernel(
        functools.partial(_sc_agg_body, with_deg),
        out_type=out_type, mesh=mesh, scratch_types=scratch,
        compiler_params=pltpu.CompilerParams(use_tc_tiling_on_sc=False),
        name="sc_agg_deg" if with_deg else "sc_agg")


_sc_agg_deg = _make_sc_agg(True)
_sc_agg = _make_sc_agg(False)

BLK = 1000
GRID = N // BLK


def _mm1_body(x_ref, w1t_ref, b1_ref, out_ref):
    out_ref[...] = (
        jnp.dot(x_ref[...], w1t_ref[...], preferred_element_type=jnp.float32)
        + b1_ref[...])


def _mm2_body(h_ref, w2t_ref, b2_ref, out_ref):
    h1 = jnp.concatenate([h_ref[0], h_ref[1]], axis=1)
    out_ref[...] = (
        jnp.dot(h1, w2t_ref[...], preferred_element_type=jnp.float32)
        + b2_ref[...])


def _layer1_body(mm_ref, agg_ref, deg_ref, out_ref):
    agg = jnp.concatenate([agg_ref[0], agg_ref[1]], axis=1)
    deg = deg_ref[0, :, 0] + deg_ref[1, :, 0]
    dinv = 1.0 / jnp.maximum(deg, 1.0)
    h = jnp.maximum(mm_ref[...] + agg * dinv[:, None], 0.0)
    out_ref[0] = h[:, :HD]
    out_ref[1] = h[:, HD:]


def _layer2_body(mm_ref, agg_ref, deg_ref, wot_ref, bo_ref,
                 out_ref, sum_s, max_s):
    i = pl.program_id(0)
    agg = jnp.concatenate([agg_ref[0], agg_ref[1]], axis=1)
    deg = deg_ref[0, :, 0] + deg_ref[1, :, 0]
    dinv = 1.0 / jnp.maximum(deg, 1.0)
    h2 = jnp.maximum(mm_ref[...] + agg * dinv[:, None], 0.0)
    ps = jnp.sum(h2, axis=0, keepdims=True)
    pm = jnp.max(h2, axis=0, keepdims=True)

    @pl.when(i == 0)
    def _():
        sum_s[...] = ps
        max_s[...] = pm

    @pl.when(i > 0)
    def _():
        sum_s[...] = sum_s[...] + ps
        max_s[...] = jnp.maximum(max_s[...], pm)

    @pl.when(i == pl.num_programs(0) - 1)
    def _():
        g = jnp.concatenate([sum_s[...] * (1.0 / N), max_s[...]], axis=1)
        out_ref[...] = (
            jnp.dot(g, wot_ref[...], preferred_element_type=jnp.float32)
            + bo_ref[...])


def _tc_mm1(x, w1t, b1):
    return pl.pallas_call(
        _mm1_body,
        grid=(GRID,),
        in_specs=[
            pl.BlockSpec((BLK, D), lambda i: (i, 0)),
            pl.BlockSpec((D, D), lambda i: (0, 0)),
            pl.BlockSpec((1, D), lambda i: (0, 0)),
        ],
        out_specs=pl.BlockSpec((BLK, D), lambda i: (i, 0)),
        out_shape=jax.ShapeDtypeStruct((N, D), jnp.float32),
    )(x, w1t, b1)


def _tc_mm2(h1s, w2t, b2):
    return pl.pallas_call(
        _mm2_body,
        grid=(GRID,),
        in_specs=[
            pl.BlockSpec((NC, BLK, HD), lambda i: (0, i, 0)),
            pl.BlockSpec((D, D), lambda i: (0, 0)),
            pl.BlockSpec((1, D), lambda i: (0, 0)),
        ],
        out_specs=pl.BlockSpec((BLK, D), lambda i: (i, 0)),
        out_shape=jax.ShapeDtypeStruct((N, D), jnp.float32),
    )(h1s, w2t, b2)


def _tc_layer1(mm, agg_p, deg_p):
    return pl.pallas_call(
        _layer1_body,
        grid=(GRID,),
        in_specs=[
            pl.BlockSpec((BLK, D), lambda i: (i, 0)),
            pl.BlockSpec((NC, BLK, HD), lambda i: (0, i, 0)),
            pl.BlockSpec((NC, BLK, 16), lambda i: (0, i, 0)),
        ],
        out_specs=pl.BlockSpec((NC, BLK, HD), lambda i: (0, i, 0)),
        out_shape=jax.ShapeDtypeStruct((NC, N, HD), jnp.float32),
    )(mm, agg_p, deg_p)


def _tc_layer2(mm, agg_p, deg_p, wot, bo):
    return pl.pallas_call(
        _layer2_body,
        grid=(GRID,),
        in_specs=[
            pl.BlockSpec((BLK, D), lambda i: (i, 0)),
            pl.BlockSpec((NC, BLK, HD), lambda i: (0, i, 0)),
            pl.BlockSpec((NC, BLK, 16), lambda i: (0, i, 0)),
            pl.BlockSpec((2 * D, CLASSES), lambda i: (0, 0)),
            pl.BlockSpec((1, CLASSES), lambda i: (0, 0)),
        ],
        out_specs=pl.BlockSpec((1, CLASSES), lambda i: (0, 0)),
        out_shape=jax.ShapeDtypeStruct((1, CLASSES), jnp.float32),
        scratch_shapes=[
            pltpu.VMEM((1, D), jnp.float32),
            pltpu.VMEM((1, D), jnp.float32),
        ],
    )(mm, agg_p, deg_p, wot, bo)


def kernel(x, edge_index, W1, b1, W2, b2, Wo, bo):
    src = edge_index[0].astype(jnp.int32)
    dst = edge_index[1].astype(jnp.int32)
    pad = E_PAD - E
    src_r = jnp.concatenate([src, jnp.zeros((pad,), jnp.int32)]).reshape(
        NS * NQ, EPQ)
    dst_r = jnp.concatenate([dst, jnp.full((pad,), N, jnp.int32)]).reshape(
        NS * NQ, EPQ)

    x2 = jnp.stack([x[:, :HD], x[:, HD:]], axis=0).reshape(NC * N, HD)

    agg1_flat, deg_flat = _sc_agg_deg(x2, src_r, dst_r)
    mm1 = _tc_mm1(x, W1.T, b1.reshape(1, D))
    agg1_p = agg1_flat.reshape(NC, N_PAD, HD)
    deg_p = deg_flat.reshape(NC, N_PAD, 16)

    h1_split = _tc_layer1(mm1, agg1_p, deg_p)

    (agg2_flat,) = _sc_agg(h1_split.reshape(NC * N, HD), src_r, dst_r)
    mm2 = _tc_mm2(h1_split, W2.T, b2.reshape(1, D))
    agg2_p = agg2_flat.reshape(NC, N_PAD, HD)

    g = _tc_layer2(mm2, agg2_p, deg_p, Wo.T, bo.reshape(1, CLASSES))
    return g.reshape(CLASSES)


# speedup vs baseline: 1.6827x; 1.6827x over previous
import functools

import jax
import jax.numpy as jnp
from jax import lax
from jax.experimental import pallas as pl
from jax.experimental.pallas import tpu as pltpu
from jax.experimental.pallas import tpu_sc as plsc

N = 10000
D = 128
HD = D // 2
E = 320000
CLASSES = 40

NC = 2
NS = 16
CH = 128
CPT = 160
E_PAD = NS * CPT * CH
N_PAD = 10240
RPT = N_PAD // NS
ROW_BLKS = RPT // CH


RQ = 2
EPQ = RQ * CH
NQ = CPT // RQ


def _sc_agg_body(with_deg, x2_hbm, src_hbm, dst_hbm, *refs):
    if with_deg:
        (acc_out, deg_out, src_v, dst_v, rows_v, ones_v, b16_v,
         acc_s, deg_s, sem) = refs
    else:
        (acc_out, src_v, dst_v, rows_v, acc_s, sem) = refs

    c = lax.axis_index("c")
    s = lax.axis_index("s")

    zf = jnp.zeros((16,), jnp.float32)
    of = jnp.ones((16,), jnp.float32)
    zb = jnp.zeros((32,), jnp.bfloat16)

    def initrow(i, carry):
        for t in range(HD // 32):
            rows_v[i, pl.ds(t * 32, 32)] = zb
        if with_deg:
            b16_v[i, pl.ds(0, 16)] = zf
        return carry

    lax.fori_loop(0, CH, initrow, 0)

    if with_deg:
        def initones(i, carry):
            ones_v[i, pl.ds(0, 16)] = of
            return carry

        lax.fori_loop(0, EPQ, initones, 0)

    def zrow(k, carry):
        r = s * RPT + k * CH
        pltpu.sync_copy(rows_v.at[pl.ds(0, CH)], acc_s.at[pl.ds(r, CH)])
        if with_deg:
            pltpu.sync_copy(b16_v, deg_s.at[pl.ds(r, CH)])
        return carry

    lax.fori_loop(0, ROW_BLKS, zrow, 0)

    pltpu.sync_copy(src_hbm.at[pl.ds(s * NQ, NQ)], src_v)
    pltpu.sync_copy(dst_hbm.at[pl.ds(s * NQ, NQ)], dst_v)

    cN = c * N

    def addrow(q, carry):
        for t in range(EPQ // 16):
            sl = pl.ds(t * 16, 16)
            src_v[q, sl] = src_v[q, sl] + cN
        return carry

    lax.fori_loop(0, NQ, addrow, 0)

    plsc.subcore_barrier()

    pltpu.async_copy(x2_hbm.at[src_v.at[0]], rows_v.at[pl.ds(0, EPQ)], sem)

    def batch(q, carry):
        b = lax.rem(q, 2) * EPQ
        ob = lax.rem(q + 1, 2) * EPQ
        pltpu.make_async_copy(
            x2_hbm.at[pl.ds(0, EPQ)], rows_v.at[pl.ds(b, EPQ)], sem).wait()

        @pl.when(q + 1 < NQ)
        def _():
            pltpu.async_copy(
                x2_hbm.at[src_v.at[q + 1]], rows_v.at[pl.ds(ob, EPQ)], sem)

        pltpu.sync_copy(rows_v.at[pl.ds(b, EPQ)], acc_s.at[dst_v.at[q]],
                        add=True)
        if with_deg:
            @pl.when((q >= c * (NQ // 2)) & (q < (c + 1) * (NQ // 2)))
            def _():
                pltpu.sync_copy(ones_v, deg_s.at[dst_v.at[q]], add=True)
        return carry

    lax.fori_loop(0, NQ, batch, 0)

    plsc.subcore_barrier()

    def wrow(k, carry):
        r = s * RPT + k * CH
        pltpu.sync_copy(acc_s.at[pl.ds(r, CH)], rows_v.at[pl.ds(0, CH)])
        pltpu.sync_copy(rows_v.at[pl.ds(0, CH)],
                        acc_out.at[pl.ds(c * N_PAD + r, CH)])
        if with_deg:
            pltpu.sync_copy(deg_s.at[pl.ds(r, CH)], b16_v)
            pltpu.sync_copy(b16_v, deg_out.at[pl.ds(c * N_PAD + r, CH)])
        return carry

    lax.fori_loop(0, ROW_BLKS, wrow, 0)


def _make_sc_agg(with_deg):
    mesh = plsc.VectorSubcoreMesh(
        core_axis_name="c", subcore_axis_name="s", num_cores=NC, num_subcores=NS)
    out_type = [jax.ShapeDtypeStruct((NC * N_PAD, HD), jnp.bfloat16)]
    scratch = [
        pltpu.VMEM((NQ, EPQ), jnp.int32),
        pltpu.VMEM((NQ, EPQ), jnp.int32),
        pltpu.VMEM((2 * EPQ, HD), jnp.bfloat16),
    ]
    if with_deg:
        out_type.append(jax.ShapeDtypeStruct((NC * N_PAD, 16), jnp.float32))
        scratch += [
            pltpu.VMEM((EPQ, 16), jnp.float32),
            pltpu.VMEM((CH, 16), jnp.float32),
        ]
    scratch += [pltpu.VMEM_SHARED((N_PAD, HD), jnp.bfloat16)]
    if with_deg:
        scratch += [pltpu.VMEM_SHARED((N_PAD, 16), jnp.float32)]
    scratch += [pltpu.SemaphoreType.DMA]
    return pl.kernel(
        functools.partial(_sc_agg_body, with_deg),
        out_type=out_type, mesh=mesh, scratch_types=scratch,
        compiler_params=pltpu.CompilerParams(use_tc_tiling_on_sc=False),
        name="sc_agg_deg" if with_deg else "sc_agg")


_sc_agg_deg = _make_sc_agg(True)
_sc_agg = _make_sc_agg(False)

BLK = 1000
GRID = N // BLK


def _mm1_body(x_ref, w1t_ref, b1_ref, out_ref):
    out_ref[...] = (
        jnp.dot(x_ref[...], w1t_ref[...], preferred_element_type=jnp.float32)
        + b1_ref[...])


def _mm2_body(h_ref, w2t_ref, b2_ref, out_ref):
    h1 = jnp.concatenate([h_ref[0], h_ref[1]], axis=1).astype(jnp.float32)
    out_ref[...] = (
        jnp.dot(h1, w2t_ref[...], preferred_element_type=jnp.float32)
        + b2_ref[...])


def _layer1_body(mm_ref, agg_ref, deg_ref, out_ref):
    agg = jnp.concatenate(
        [agg_ref[0], agg_ref[1]], axis=1).astype(jnp.float32)
    deg = deg_ref[0, :, 0] + deg_ref[1, :, 0]
    dinv = 1.0 / jnp.maximum(deg, 1.0)
    h = jnp.maximum(mm_ref[...] + agg * dinv[:, None], 0.0)
    hb = h.astype(jnp.bfloat16)
    out_ref[0] = hb[:, :HD]
    out_ref[1] = hb[:, HD:]


def _layer2_body(mm_ref, agg_ref, deg_ref, wot_ref, bo_ref,
                 out_ref, sum_s, max_s):
    i = pl.program_id(0)
    agg = jnp.concatenate(
        [agg_ref[0], agg_ref[1]], axis=1).astype(jnp.float32)
    deg = deg_ref[0, :, 0] + deg_ref[1, :, 0]
    dinv = 1.0 / jnp.maximum(deg, 1.0)
    h2 = jnp.maximum(mm_ref[...] + agg * dinv[:, None], 0.0)
    ps = jnp.sum(h2, axis=0, keepdims=True)
    pm = jnp.max(h2, axis=0, keepdims=True)

    @pl.when(i == 0)
    def _():
        sum_s[...] = ps
        max_s[...] = pm

    @pl.when(i > 0)
    def _():
        sum_s[...] = sum_s[...] + ps
        max_s[...] = jnp.maximum(max_s[...], pm)

    @pl.when(i == pl.num_programs(0) - 1)
    def _():
        g = jnp.concatenate([sum_s[...] * (1.0 / N), max_s[...]], axis=1)
        out_ref[...] = (
            jnp.dot(g, wot_ref[...], preferred_element_type=jnp.float32)
            + bo_ref[...])


def _tc_mm1(x, w1t, b1):
    return pl.pallas_call(
        _mm1_body,
        grid=(GRID,),
        in_specs=[
            pl.BlockSpec((BLK, D), lambda i: (i, 0)),
            pl.BlockSpec((D, D), lambda i: (0, 0)),
            pl.BlockSpec((1, D), lambda i: (0, 0)),
        ],
        out_specs=pl.BlockSpec((BLK, D), lambda i: (i, 0)),
        out_shape=jax.ShapeDtypeStruct((N, D), jnp.float32),
    )(x, w1t, b1)


def _tc_mm2(h1s, w2t, b2):
    return pl.pallas_call(
        _mm2_body,
        grid=(GRID,),
        in_specs=[
            pl.BlockSpec((NC, BLK, HD), lambda i: (0, i, 0)),
            pl.BlockSpec((D, D), lambda i: (0, 0)),
            pl.BlockSpec((1, D), lambda i: (0, 0)),
        ],
        out_specs=pl.BlockSpec((BLK, D), lambda i: (i, 0)),
        out_shape=jax.ShapeDtypeStruct((N, D), jnp.float32),
    )(h1s, w2t, b2)


def _tc_layer1(mm, agg_p, deg_p):
    return pl.pallas_call(
        _layer1_body,
        grid=(GRID,),
        in_specs=[
            pl.BlockSpec((BLK, D), lambda i: (i, 0)),
            pl.BlockSpec((NC, BLK, HD), lambda i: (0, i, 0)),
            pl.BlockSpec((NC, BLK, 16), lambda i: (0, i, 0)),
        ],
        out_specs=pl.BlockSpec((NC, BLK, HD), lambda i: (0, i, 0)),
        out_shape=jax.ShapeDtypeStruct((NC, N, HD), jnp.bfloat16),
    )(mm, agg_p, deg_p)


def _tc_layer2(mm, agg_p, deg_p, wot, bo):
    return pl.pallas_call(
        _layer2_body,
        grid=(GRID,),
        in_specs=[
            pl.BlockSpec((BLK, D), lambda i: (i, 0)),
            pl.BlockSpec((NC, BLK, HD), lambda i: (0, i, 0)),
            pl.BlockSpec((NC, BLK, 16), lambda i: (0, i, 0)),
            pl.BlockSpec((2 * D, CLASSES), lambda i: (0, 0)),
            pl.BlockSpec((1, CLASSES), lambda i: (0, 0)),
        ],
        out_specs=pl.BlockSpec((1, CLASSES), lambda i: (0, 0)),
        out_shape=jax.ShapeDtypeStruct((1, CLASSES), jnp.float32),
        scratch_shapes=[
            pltpu.VMEM((1, D), jnp.float32),
            pltpu.VMEM((1, D), jnp.float32),
        ],
    )(mm, agg_p, deg_p, wot, bo)


def kernel(x, edge_index, W1, b1, W2, b2, Wo, bo):
    src = edge_index[0].astype(jnp.int32)
    dst = edge_index[1].astype(jnp.int32)
    pad = E_PAD - E
    src_r = jnp.concatenate([src, jnp.zeros((pad,), jnp.int32)]).reshape(
        NS * NQ, EPQ)
    dst_r = jnp.concatenate([dst, jnp.full((pad,), N, jnp.int32)]).reshape(
        NS * NQ, EPQ)

    x2 = jnp.stack([x[:, :HD], x[:, HD:]], axis=0).reshape(
        NC * N, HD).astype(jnp.bfloat16)

    agg1_flat, deg_flat = _sc_agg_deg(x2, src_r, dst_r)
    mm1 = _tc_mm1(x, W1.T, b1.reshape(1, D))
    agg1_p = agg1_flat.reshape(NC, N_PAD, HD)
    deg_p = deg_flat.reshape(NC, N_PAD, 16)

    h1_split = _tc_layer1(mm1, agg1_p, deg_p)

    (agg2_flat,) = _sc_agg(h1_split.reshape(NC * N, HD), src_r, dst_r)
    mm2 = _tc_mm2(h1_split, W2.T, b2.reshape(1, D))
    agg2_p = agg2_flat.reshape(NC, N_PAD, HD)

    g = _tc_layer2(mm2, agg2_p, deg_p, Wo.T, bo.reshape(1, CLASSES))
    return g.reshape(CLASSES)


# speedup vs baseline: 1.8161x; 1.0793x over previous
import functools

import jax
import jax.numpy as jnp
from jax import lax
from jax.experimental import pallas as pl
from jax.experimental.pallas import tpu as pltpu
from jax.experimental.pallas import tpu_sc as plsc

N = 10000
D = 128
HD = D // 2
E = 320000
CLASSES = 40

NC = 2
NS = 16
CH = 128
CPT = 160
E_PAD = NS * CPT * CH
N_PAD = 10240
RPT = N_PAD // NS
ROW_BLKS = RPT // CH


RQ = 4
EPQ = RQ * CH
NQ = CPT // RQ


def _sc_agg_body(with_deg, x2_hbm, src_hbm, dst_hbm, *refs):
    if with_deg:
        (acc_out, deg_out, src_v, dst_v, rows_v, ones_v, b16_v,
         acc_s, deg_s, sem) = refs
    else:
        (acc_out, src_v, dst_v, rows_v, acc_s, sem) = refs

    c = lax.axis_index("c")
    s = lax.axis_index("s")

    zf = jnp.zeros((16,), jnp.float32)
    of = jnp.ones((16,), jnp.float32)
    zb = jnp.zeros((32,), jnp.bfloat16)

    def initrow(i, carry):
        for t in range(HD // 32):
            rows_v[i, pl.ds(t * 32, 32)] = zb
        if with_deg:
            b16_v[i, pl.ds(0, 16)] = zf
        return carry

    lax.fori_loop(0, CH, initrow, 0)

    if with_deg:
        def initones(i, carry):
            ones_v[i, pl.ds(0, 16)] = of
            return carry

        lax.fori_loop(0, EPQ, initones, 0)

    def zrow(k, carry):
        r = s * RPT + k * CH
        pltpu.sync_copy(rows_v.at[pl.ds(0, CH)], acc_s.at[pl.ds(r, CH)])
        if with_deg:
            pltpu.sync_copy(b16_v, deg_s.at[pl.ds(r, CH)])
        return carry

    lax.fori_loop(0, ROW_BLKS, zrow, 0)

    pltpu.sync_copy(src_hbm.at[pl.ds(s * NQ, NQ)], src_v)
    pltpu.sync_copy(dst_hbm.at[pl.ds(s * NQ, NQ)], dst_v)

    cN = c * N

    def addrow(q, carry):
        for t in range(EPQ // 16):
            sl = pl.ds(t * 16, 16)
            src_v[q, sl] = src_v[q, sl] + cN
        return carry

    lax.fori_loop(0, NQ, addrow, 0)

    plsc.subcore_barrier()

    pltpu.async_copy(x2_hbm.at[src_v.at[0]], rows_v.at[pl.ds(0, EPQ)], sem)

    def batch(q, carry):
        b = lax.rem(q, 2) * EPQ
        ob = lax.rem(q + 1, 2) * EPQ
        pltpu.make_async_copy(
            x2_hbm.at[pl.ds(0, EPQ)], rows_v.at[pl.ds(b, EPQ)], sem).wait()

        @pl.when(q + 1 < NQ)
        def _():
            pltpu.async_copy(
                x2_hbm.at[src_v.at[q + 1]], rows_v.at[pl.ds(ob, EPQ)], sem)

        pltpu.sync_copy(rows_v.at[pl.ds(b, EPQ)], acc_s.at[dst_v.at[q]],
                        add=True)
        if with_deg:
            @pl.when((q >= c * (NQ // 2)) & (q < (c + 1) * (NQ // 2)))
            def _():
                pltpu.sync_copy(ones_v, deg_s.at[dst_v.at[q]], add=True)
        return carry

    lax.fori_loop(0, NQ, batch, 0)

    plsc.subcore_barrier()

    def wrow(k, carry):
        r = s * RPT + k * CH
        pltpu.sync_copy(acc_s.at[pl.ds(r, CH)], rows_v.at[pl.ds(0, CH)])
        pltpu.sync_copy(rows_v.at[pl.ds(0, CH)],
                        acc_out.at[pl.ds(c * N_PAD + r, CH)])
        if with_deg:
            pltpu.sync_copy(deg_s.at[pl.ds(r, CH)], b16_v)
            pltpu.sync_copy(b16_v, deg_out.at[pl.ds(c * N_PAD + r, CH)])
        return carry

    lax.fori_loop(0, ROW_BLKS, wrow, 0)


def _make_sc_agg(with_deg):
    mesh = plsc.VectorSubcoreMesh(
        core_axis_name="c", subcore_axis_name="s", num_cores=NC, num_subcores=NS)
    out_type = [jax.ShapeDtypeStruct((NC * N_PAD, HD), jnp.bfloat16)]
    scratch = [
        pltpu.VMEM((NQ, EPQ), jnp.int32),
        pltpu.VMEM((NQ, EPQ), jnp.int32),
        pltpu.VMEM((2 * EPQ, HD), jnp.bfloat16),
    ]
    if with_deg:
        out_type.append(jax.ShapeDtypeStruct((NC * N_PAD, 16), jnp.float32))
        scratch += [
            pltpu.VMEM((EPQ, 16), jnp.float32),
            pltpu.VMEM((CH, 16), jnp.float32),
        ]
    scratch += [pltpu.VMEM_SHARED((N_PAD, HD), jnp.bfloat16)]
    if with_deg:
        scratch += [pltpu.VMEM_SHARED((N_PAD, 16), jnp.float32)]
    scratch += [pltpu.SemaphoreType.DMA]
    return pl.kernel(
        functools.partial(_sc_agg_body, with_deg),
        out_type=out_type, mesh=mesh, scratch_types=scratch,
        compiler_params=pltpu.CompilerParams(use_tc_tiling_on_sc=False),
        name="sc_agg_deg" if with_deg else "sc_agg")


_sc_agg_deg = _make_sc_agg(True)
_sc_agg = _make_sc_agg(False)

BLK = 1000
GRID = N // BLK


def _mm1_body(x_ref, w1t_ref, b1_ref, out_ref):
    out_ref[...] = (
        jnp.dot(x_ref[...], w1t_ref[...], preferred_element_type=jnp.float32)
        + b1_ref[...])


def _mm2_body(h_ref, w2t_ref, b2_ref, out_ref):
    h1 = jnp.concatenate([h_ref[0], h_ref[1]], axis=1).astype(jnp.float32)
    out_ref[...] = (
        jnp.dot(h1, w2t_ref[...], preferred_element_type=jnp.float32)
        + b2_ref[...])


def _layer1_body(mm_ref, agg_ref, deg_ref, out_ref):
    agg = jnp.concatenate(
        [agg_ref[0], agg_ref[1]], axis=1).astype(jnp.float32)
    deg = deg_ref[0, :, 0] + deg_ref[1, :, 0]
    dinv = 1.0 / jnp.maximum(deg, 1.0)
    h = jnp.maximum(mm_ref[...] + agg * dinv[:, None], 0.0)
    hb = h.astype(jnp.bfloat16)
    out_ref[0] = hb[:, :HD]
    out_ref[1] = hb[:, HD:]


def _layer2_body(mm_ref, agg_ref, deg_ref, wot_ref, bo_ref,
                 out_ref, sum_s, max_s):
    i = pl.program_id(0)
    agg = jnp.concatenate(
        [agg_ref[0], agg_ref[1]], axis=1).astype(jnp.float32)
    deg = deg_ref[0, :, 0] + deg_ref[1, :, 0]
    dinv = 1.0 / jnp.maximum(deg, 1.0)
    h2 = jnp.maximum(mm_ref[...] + agg * dinv[:, None], 0.0)
    ps = jnp.sum(h2, axis=0, keepdims=True)
    pm = jnp.max(h2, axis=0, keepdims=True)

    @pl.when(i == 0)
    def _():
        sum_s[...] = ps
        max_s[...] = pm

    @pl.when(i > 0)
    def _():
        sum_s[...] = sum_s[...] + ps
        max_s[...] = jnp.maximum(max_s[...], pm)

    @pl.when(i == pl.num_programs(0) - 1)
    def _():
        g = jnp.concatenate([sum_s[...] * (1.0 / N), max_s[...]], axis=1)
        out_ref[...] = (
            jnp.dot(g, wot_ref[...], preferred_element_type=jnp.float32)
            + bo_ref[...])


def _tc_mm1(x, w1t, b1):
    return pl.pallas_call(
        _mm1_body,
        grid=(GRID,),
        in_specs=[
            pl.BlockSpec((BLK, D), lambda i: (i, 0)),
            pl.BlockSpec((D, D), lambda i: (0, 0)),
            pl.BlockSpec((1, D), lambda i: (0, 0)),
        ],
        out_specs=pl.BlockSpec((BLK, D), lambda i: (i, 0)),
        out_shape=jax.ShapeDtypeStruct((N, D), jnp.float32),
    )(x, w1t, b1)


def _tc_mm2(h1s, w2t, b2):
    return pl.pallas_call(
        _mm2_body,
        grid=(GRID,),
        in_specs=[
            pl.BlockSpec((NC, BLK, HD), lambda i: (0, i, 0)),
            pl.BlockSpec((D, D), lambda i: (0, 0)),
            pl.BlockSpec((1, D), lambda i: (0, 0)),
        ],
        out_specs=pl.BlockSpec((BLK, D), lambda i: (i, 0)),
        out_shape=jax.ShapeDtypeStruct((N, D), jnp.float32),
    )(h1s, w2t, b2)


def _tc_layer1(mm, agg_p, deg_p):
    return pl.pallas_call(
        _layer1_body,
        grid=(GRID,),
        in_specs=[
            pl.BlockSpec((BLK, D), lambda i: (i, 0)),
            pl.BlockSpec((NC, BLK, HD), lambda i: (0, i, 0)),
            pl.BlockSpec((NC, BLK, 16), lambda i: (0, i, 0)),
        ],
        out_specs=pl.BlockSpec((NC, BLK, HD), lambda i: (0, i, 0)),
        out_shape=jax.ShapeDtypeStruct((NC, N, HD), jnp.bfloat16),
    )(mm, agg_p, deg_p)


def _tc_layer2(mm, agg_p, deg_p, wot, bo):
    return pl.pallas_call(
        _layer2_body,
        grid=(GRID,),
        in_specs=[
            pl.BlockSpec((BLK, D), lambda i: (i, 0)),
            pl.BlockSpec((NC, BLK, HD), lambda i: (0, i, 0)),
            pl.BlockSpec((NC, BLK, 16), lambda i: (0, i, 0)),
            pl.BlockSpec((2 * D, CLASSES), lambda i: (0, 0)),
            pl.BlockSpec((1, CLASSES), lambda i: (0, 0)),
        ],
        out_specs=pl.BlockSpec((1, CLASSES), lambda i: (0, 0)),
        out_shape=jax.ShapeDtypeStruct((1, CLASSES), jnp.float32),
        scratch_shapes=[
            pltpu.VMEM((1, D), jnp.float32),
            pltpu.VMEM((1, D), jnp.float32),
        ],
    )(mm, agg_p, deg_p, wot, bo)


def kernel(x, edge_index, W1, b1, W2, b2, Wo, bo):
    src = edge_index[0].astype(jnp.int32)
    dst = edge_index[1].astype(jnp.int32)
    pad = E_PAD - E
    src_r = jnp.concatenate([src, jnp.zeros((pad,), jnp.int32)]).reshape(
        NS * NQ, EPQ)
    dst_r = jnp.concatenate([dst, jnp.full((pad,), N, jnp.int32)]).reshape(
        NS * NQ, EPQ)

    x2 = jnp.stack([x[:, :HD], x[:, HD:]], axis=0).reshape(
        NC * N, HD).astype(jnp.bfloat16)

    agg1_flat, deg_flat = _sc_agg_deg(x2, src_r, dst_r)
    mm1 = _tc_mm1(x, W1.T, b1.reshape(1, D))
    agg1_p = agg1_flat.reshape(NC, N_PAD, HD)
    deg_p = deg_flat.reshape(NC, N_PAD, 16)

    h1_split = _tc_layer1(mm1, agg1_p, deg_p)

    (agg2_flat,) = _sc_agg(h1_split.reshape(NC * N, HD), src_r, dst_r)
    mm2 = _tc_mm2(h1_split, W2.T, b2.reshape(1, D))
    agg2_p = agg2_flat.reshape(NC, N_PAD, HD)

    g = _tc_layer2(mm2, agg2_p, deg_p, Wo.T, bo.reshape(1, CLASSES))
    return g.reshape(CLASSES)
